# Initial kernel scaffold; baseline (speedup 1.0000x reference)
import jax
import jax.numpy as jnp
from jax.experimental import pallas as pl


def kernel(x, edge_index, W1, b1, W2, b2):
    raise NotImplementedError("write your pallas kernel here")


# speedup vs baseline: 18.9966x; 18.9966x over previous
import functools

import jax
import jax.numpy as jnp
from jax import lax
from jax.experimental import pallas as pl
from jax.experimental.pallas import tpu as pltpu
from jax.experimental.pallas import tpu_sc as plsc

N = 10000
E = 320000
D = 128
H = 64

NC = 2
NS = 16
NW = NC * NS
WIN = 128
E_PAD = 327680
EPT = E_PAD // NW
NWIN = EPT // WIN
N_PAD = 10112
RPT = N_PAD // NS
F2 = 16

_MESH = plsc.VectorSubcoreMesh(
    core_axis_name="c", subcore_axis_name="s", num_cores=NC, num_subcores=NS)


def _make_agg(F, name):
  @functools.partial(
      pl.kernel,
      out_type=jax.ShapeDtypeStruct((NC, N_PAD, F), jnp.float32),
      mesh=_MESH,
      scratch_types=[
          pltpu.VMEM((WIN,), jnp.int32),
          pltpu.VMEM((WIN,), jnp.int32),
          pltpu.VMEM((WIN, F), jnp.float32),
          pltpu.VMEM_SHARED((N_PAD, F), jnp.float32),
          pltpu.SemaphoreType.DMA,
      ],
      compiler_params=pltpu.CompilerParams(use_tc_tiling_on_sc=False),
      name=name,
  )
  def agg(y_hbm, src_hbm, dst_hbm, z_hbm, out_hbm, src_v, dst_v, rows_v,
          acc_sh, sem):
    c = lax.axis_index("c")
    s = lax.axis_index("s")
    wid = c * NS + s
    pltpu.sync_copy(z_hbm, acc_sh.at[pl.ds(s * RPT, RPT)])
    plsc.subcore_barrier()

    def win(w, carry):
      base = wid * EPT + w * WIN
      pltpu.sync_copy(src_hbm.at[pl.ds(base, WIN)], src_v)
      pltpu.sync_copy(dst_hbm.at[pl.ds(base, WIN)], dst_v)
      pltpu.async_copy(y_hbm.at[src_v], rows_v, sem).wait()
      pltpu.sync_copy(rows_v, acc_sh.at[dst_v], add=True)
      return carry

    lax.fori_loop(0, NWIN, win, 0)
    plsc.subcore_barrier()
    pltpu.sync_copy(acc_sh.at[pl.ds(s * RPT, RPT)],
                    out_hbm.at[c].at[pl.ds(s * RPT, RPT)])

  return agg


@functools.partial(
    pl.kernel,
    out_type=jax.ShapeDtypeStruct((NC, N_PAD, F2), jnp.float32),
    mesh=_MESH,
    scratch_types=[
        pltpu.VMEM((WIN,), jnp.int32),
        pltpu.VMEM((WIN, F2), jnp.float32),
        pltpu.VMEM_SHARED((N_PAD, F2), jnp.float32),
    ],
    compiler_params=pltpu.CompilerParams(use_tc_tiling_on_sc=False),
    name="gcn_deg_sc",
)
def _deg_sc(dst_hbm, ones_hbm, z_hbm, out_hbm, dst_v, ones_v, acc_sh):
  c = lax.axis_index("c")
  s = lax.axis_index("s")
  wid = c * NS + s
  pltpu.sync_copy(ones_hbm, ones_v)
  pltpu.sync_copy(z_hbm, acc_sh.at[pl.ds(s * RPT, RPT)])
  plsc.subcore_barrier()

  def win(w, carry):
    base = wid * EPT + w * WIN
    pltpu.sync_copy(dst_hbm.at[pl.ds(base, WIN)], dst_v)
    pltpu.sync_copy(ones_v, acc_sh.at[dst_v], add=True)
    return carry

  lax.fori_loop(0, NWIN, win, 0)
  plsc.subcore_barrier()
  pltpu.sync_copy(acc_sh.at[pl.ds(s * RPT, RPT)],
                  out_hbm.at[c].at[pl.ds(s * RPT, RPT)])


_agg64 = _make_agg(H, "gcn_agg64_sc")
_agg16 = _make_agg(F2, "gcn_agg16_sc")

_RB = 1000
_GRID = (N // _RB,)


def _tc1_body(x_ref, w1_ref, dp_ref, xw_ref, y1_ref, dinv_ref):
  deg = 1.0 + dp_ref[0] + dp_ref[1]
  dinv = lax.rsqrt(deg)
  xw = jnp.dot(x_ref[...], w1_ref[...], preferred_element_type=jnp.float32)
  d0 = dinv[:, 0:1]
  xw_ref[...] = xw
  y1_ref[...] = xw * d0
  dinv_ref[...] = dinv


_tc1 = pl.pallas_call(
    _tc1_body,
    grid=_GRID,
    in_specs=[
        pl.BlockSpec((_RB, D), lambda i: (i, 0)),
        pl.BlockSpec((D, H), lambda i: (0, 0)),
        pl.BlockSpec((NC, _RB, F2), lambda i: (0, i, 0)),
    ],
    out_specs=[
        pl.BlockSpec((_RB, H), lambda i: (i, 0)),
        pl.BlockSpec((_RB, H), lambda i: (i, 0)),
        pl.BlockSpec((_RB, F2), lambda i: (i, 0)),
    ],
    out_shape=[
        jax.ShapeDtypeStruct((N, H), jnp.float32),
        jax.ShapeDtypeStruct((N, H), jnp.float32),
        jax.ShapeDtypeStruct((N, F2), jnp.float32),
    ],
)


def _tc2_body(a1_ref, xw_ref, dinv_ref, b1_ref, w2_ref, hw2_ref, y2_ref):
  d0 = dinv_ref[:, 0:1]
  agg = a1_ref[0] + a1_ref[1]
  out1 = d0 * agg + (d0 * d0) * xw_ref[...] + b1_ref[...]
  h = jnp.maximum(out1, 0.0)
  hw2 = jnp.dot(h, w2_ref[...], preferred_element_type=jnp.float32)
  hw2_ref[...] = hw2
  y2_ref[...] = hw2 * d0


_tc2 = pl.pallas_call(
    _tc2_body,
    grid=_GRID,
    in_specs=[
        pl.BlockSpec((NC, _RB, H), lambda i: (0, i, 0)),
        pl.BlockSpec((_RB, H), lambda i: (i, 0)),
        pl.BlockSpec((_RB, F2), lambda i: (i, 0)),
        pl.BlockSpec((1, H), lambda i: (0, 0)),
        pl.BlockSpec((H, F2), lambda i: (0, 0)),
    ],
    out_specs=[
        pl.BlockSpec((_RB, F2), lambda i: (i, 0)),
        pl.BlockSpec((_RB, F2), lambda i: (i, 0)),
    ],
    out_shape=[
        jax.ShapeDtypeStruct((N, F2), jnp.float32),
        jax.ShapeDtypeStruct((N, F2), jnp.float32),
    ],
)


def _tc3_body(a2_ref, hw2_ref, dinv_ref, b2_ref, out_ref):
  d0 = dinv_ref[:, 0:1]
  agg = a2_ref[0] + a2_ref[1]
  out_ref[...] = d0 * agg + (d0 * d0) * hw2_ref[...] + b2_ref[...]


_tc3 = pl.pallas_call(
    _tc3_body,
    grid=_GRID,
    in_specs=[
        pl.BlockSpec((NC, _RB, F2), lambda i: (0, i, 0)),
        pl.BlockSpec((_RB, F2), lambda i: (i, 0)),
        pl.BlockSpec((_RB, F2), lambda i: (i, 0)),
        pl.BlockSpec((1, F2), lambda i: (0, 0)),
    ],
    out_specs=pl.BlockSpec((_RB, F2), lambda i: (i, 0)),
    out_shape=jax.ShapeDtypeStruct((N, F2), jnp.float32),
)


def kernel(x, edge_index, W1, b1, W2, b2):
  ei = edge_index.astype(jnp.int32)
  n_extra = E_PAD - E
  pad_src = (jnp.arange(n_extra, dtype=jnp.int32) % 9973)
  pad_dst = N + (jnp.arange(n_extra, dtype=jnp.int32) % (N_PAD - N))
  src = jnp.concatenate([ei[0], pad_src])
  dst = jnp.concatenate([ei[1], pad_dst])

  ones_w = jnp.ones((WIN, F2), jnp.float32)
  z16 = jnp.zeros((RPT, F2), jnp.float32)
  z64 = jnp.zeros((RPT, H), jnp.float32)

  dp = _deg_sc(dst, ones_w, z16)
  xw, y1, dinv = _tc1(x, W1, dp[:, :N])
  a1 = _agg64(y1, src, dst, z64)
  b1r = b1.reshape(1, H)
  w2p = jnp.zeros((H, F2), jnp.float32).at[:, :2].set(W2)
  hw2, y2 = _tc2(a1[:, :N], xw, dinv, b1r, w2p)
  a2 = _agg16(y2, src, dst, z16)
  b2p = jnp.zeros((1, F2), jnp.float32).at[0, :2].set(b2)
  out16 = _tc3(a2[:, :N], hw2, dinv, b2p)
  return out16[:, :2]


# speedup vs baseline: 38.3294x; 2.0177x over previous
import functools

import jax
import jax.numpy as jnp
from jax import lax
from jax.experimental import pallas as pl
from jax.experimental.pallas import tpu as pltpu
from jax.experimental.pallas import tpu_sc as plsc

N = 10000
E = 320000
D = 128
H = 64

NC = 2
NS = 16
NW = NC * NS
WIN = 128
E_PAD = 327680
EPT = E_PAD // NW
NWIN = EPT // WIN
N_PAD = 10112
RPT = N_PAD // NS
F2 = 16

_MESH = plsc.VectorSubcoreMesh(
    core_axis_name="c", subcore_axis_name="s", num_cores=NC, num_subcores=NS)


def _make_agg(F, name):
  @functools.partial(
      pl.kernel,
      out_type=jax.ShapeDtypeStruct((NC, N_PAD, F), jnp.float32),
      mesh=_MESH,
      scratch_types=[
          pltpu.VMEM((NWIN, WIN), jnp.int32),
          pltpu.VMEM((NWIN, WIN), jnp.int32),
          pltpu.VMEM((WIN, F), jnp.float32),
          pltpu.VMEM((WIN, F), jnp.float32),
          pltpu.VMEM_SHARED((N_PAD, F), jnp.float32),
          pltpu.SemaphoreType.DMA,
          pltpu.SemaphoreType.DMA,
      ],
      compiler_params=pltpu.CompilerParams(use_tc_tiling_on_sc=False),
      name=name,
  )
  def agg(y_hbm, srcw_hbm, dstw_hbm, z_hbm, out_hbm, src_i, dst_i, rows0,
          rows1, acc_sh, g0, g1):
    c = lax.axis_index("c")
    s = lax.axis_index("s")
    wid = c * NS + s
    wbase = wid * NWIN
    pltpu.sync_copy(srcw_hbm.at[pl.ds(wbase, NWIN)], src_i)
    pltpu.sync_copy(dstw_hbm.at[pl.ds(wbase, NWIN)], dst_i)
    pltpu.sync_copy(z_hbm, acc_sh.at[pl.ds(s * RPT, RPT)])
    plsc.subcore_barrier()

    pltpu.async_copy(y_hbm.at[src_i.at[0]], rows0, g0)
    pltpu.async_copy(y_hbm.at[src_i.at[1]], rows1, g1)

    def win(i, carry):
      w0 = 2 * i
      w1 = w0 + 1
      pltpu.make_async_copy(y_hbm.at[src_i.at[w0]], rows0, g0).wait()
      pltpu.sync_copy(rows0, acc_sh.at[dst_i.at[w0]], add=True)

      @pl.when(w0 + 2 < NWIN)
      def _():
        pltpu.async_copy(y_hbm.at[src_i.at[w0 + 2]], rows0, g0)

      pltpu.make_async_copy(y_hbm.at[src_i.at[w1]], rows1, g1).wait()
      pltpu.sync_copy(rows1, acc_sh.at[dst_i.at[w1]], add=True)

      @pl.when(w1 + 2 < NWIN)
      def _():
        pltpu.async_copy(y_hbm.at[src_i.at[w1 + 2]], rows1, g1)

      return carry

    lax.fori_loop(0, NWIN // 2, win, 0)
    plsc.subcore_barrier()
    pltpu.sync_copy(acc_sh.at[pl.ds(s * RPT, RPT)],
                    out_hbm.at[c].at[pl.ds(s * RPT, RPT)])

  return agg


@functools.partial(
    pl.kernel,
    out_type=jax.ShapeDtypeStruct((NC, N_PAD, F2), jnp.float32),
    mesh=_MESH,
    scratch_types=[
        pltpu.VMEM((NWIN, WIN), jnp.int32),
        pltpu.VMEM((WIN, F2), jnp.float32),
        pltpu.VMEM_SHARED((N_PAD, F2), jnp.float32),
        pltpu.SemaphoreType.DMA,
    ],
    compiler_params=pltpu.CompilerParams(use_tc_tiling_on_sc=False),
    name="gcn_deg_sc",
)
def _deg_sc(dstw_hbm, ones_hbm, z_hbm, out_hbm, dst_i, ones_v, acc_sh, sem):
  c = lax.axis_index("c")
  s = lax.axis_index("s")
  wid = c * NS + s
  pltpu.sync_copy(dstw_hbm.at[pl.ds(wid * NWIN, NWIN)], dst_i)
  pltpu.sync_copy(ones_hbm, ones_v)
  pltpu.sync_copy(z_hbm, acc_sh.at[pl.ds(s * RPT, RPT)])
  plsc.subcore_barrier()

  def grp(g, carry):
    base = g * 8
    for j in range(8):
      pltpu.async_copy(ones_v, acc_sh.at[dst_i.at[base + j]], sem, add=True)
    for j in range(8):
      pltpu.make_async_copy(ones_v, acc_sh.at[dst_i.at[base + j]], sem).wait()
    return carry

  lax.fori_loop(0, NWIN // 8, grp, 0)
  plsc.subcore_barrier()
  pltpu.sync_copy(acc_sh.at[pl.ds(s * RPT, RPT)],
                  out_hbm.at[c].at[pl.ds(s * RPT, RPT)])


_agg64 = _make_agg(H, "gcn_agg64_sc")
_agg16 = _make_agg(F2, "gcn_agg16_sc")

_RB = 1000
_GRID = (N // _RB,)


def _tc1_body(x_ref, w1_ref, dp_ref, xw_ref, y1_ref, dinv_ref):
  deg = 1.0 + dp_ref[0] + dp_ref[1]
  dinv = lax.rsqrt(deg)
  xw = jnp.dot(x_ref[...], w1_ref[...], preferred_element_type=jnp.float32)
  d0 = dinv[:, 0:1]
  xw_ref[...] = xw
  y1_ref[...] = xw * d0
  dinv_ref[...] = dinv


_tc1 = pl.pallas_call(
    _tc1_body,
    grid=_GRID,
    in_specs=[
        pl.BlockSpec((_RB, D), lambda i: (i, 0)),
        pl.BlockSpec((D, H), lambda i: (0, 0)),
        pl.BlockSpec((NC, _RB, F2), lambda i: (0, i, 0)),
    ],
    out_specs=[
        pl.BlockSpec((_RB, H), lambda i: (i, 0)),
        pl.BlockSpec((_RB, H), lambda i: (i, 0)),
        pl.BlockSpec((_RB, F2), lambda i: (i, 0)),
    ],
    out_shape=[
        jax.ShapeDtypeStruct((N, H), jnp.float32),
        jax.ShapeDtypeStruct((N, H), jnp.float32),
        jax.ShapeDtypeStruct((N, F2), jnp.float32),
    ],
)


def _tc2_body(a1_ref, xw_ref, dinv_ref, b1_ref, w2_ref, hw2_ref, y2_ref):
  d0 = dinv_ref[:, 0:1]
  agg = a1_ref[0] + a1_ref[1]
  out1 = d0 * agg + (d0 * d0) * xw_ref[...] + b1_ref[...]
  h = jnp.maximum(out1, 0.0)
  hw2 = jnp.dot(h, w2_ref[...], preferred_element_type=jnp.float32)
  hw2_ref[...] = hw2
  y2_ref[...] = hw2 * d0


_tc2 = pl.pallas_call(
    _tc2_body,
    grid=_GRID,
    in_specs=[
        pl.BlockSpec((NC, _RB, H), lambda i: (0, i, 0)),
        pl.BlockSpec((_RB, H), lambda i: (i, 0)),
        pl.BlockSpec((_RB, F2), lambda i: (i, 0)),
        pl.BlockSpec((1, H), lambda i: (0, 0)),
        pl.BlockSpec((H, F2), lambda i: (0, 0)),
    ],
    out_specs=[
        pl.BlockSpec((_RB, F2), lambda i: (i, 0)),
        pl.BlockSpec((_RB, F2), lambda i: (i, 0)),
    ],
    out_shape=[
        jax.ShapeDtypeStruct((N, F2), jnp.float32),
        jax.ShapeDtypeStruct((N, F2), jnp.float32),
    ],
)


def _tc3_body(a2_ref, hw2_ref, dinv_ref, b2_ref, out_ref):
  d0 = dinv_ref[:, 0:1]
  agg = a2_ref[0] + a2_ref[1]
  out_ref[...] = d0 * agg + (d0 * d0) * hw2_ref[...] + b2_ref[...]


_tc3 = pl.pallas_call(
    _tc3_body,
    grid=_GRID,
    in_specs=[
        pl.BlockSpec((NC, _RB, F2), lambda i: (0, i, 0)),
        pl.BlockSpec((_RB, F2), lambda i: (i, 0)),
        pl.BlockSpec((_RB, F2), lambda i: (i, 0)),
        pl.BlockSpec((1, F2), lambda i: (0, 0)),
    ],
    out_specs=pl.BlockSpec((_RB, F2), lambda i: (i, 0)),
    out_shape=jax.ShapeDtypeStruct((N, F2), jnp.float32),
)


def kernel(x, edge_index, W1, b1, W2, b2):
  ei = edge_index.astype(jnp.int32)
  n_extra = E_PAD - E
  pad_src = (jnp.arange(n_extra, dtype=jnp.int32) % 9973)
  pad_dst = N + (jnp.arange(n_extra, dtype=jnp.int32) % (N_PAD - N))
  src = jnp.concatenate([ei[0], pad_src]).reshape(E_PAD // WIN, WIN)
  dst = jnp.concatenate([ei[1], pad_dst]).reshape(E_PAD // WIN, WIN)

  ones_w = jnp.ones((WIN, F2), jnp.float32)
  z16 = jnp.zeros((RPT, F2), jnp.float32)
  z64 = jnp.zeros((RPT, H), jnp.float32)

  dp = _deg_sc(dst, ones_w, z16)
  xw, y1, dinv = _tc1(x, W1, dp[:, :N])
  a1 = _agg64(y1, src, dst, z64)
  b1r = b1.reshape(1, H)
  w2p = jnp.zeros((H, F2), jnp.float32).at[:, :2].set(W2)
  hw2, y2 = _tc2(a1[:, :N], xw, dinv, b1r, w2p)
  a2 = _agg16(y2, src, dst, z16)
  b2p = jnp.zeros((1, F2), jnp.float32).at[0, :2].set(b2)
  out16 = _tc3(a2[:, :N], hw2, dinv, b2p)
  return out16[:, :2]


# speedup vs baseline: 41.6882x; 1.0876x over previous
import functools

import jax
import jax.numpy as jnp
from jax import lax
from jax.experimental import pallas as pl
from jax.experimental.pallas import tpu as pltpu
from jax.experimental.pallas import tpu_sc as plsc

N = 10000
E = 320000
D = 128
H = 64

NC = 2
NS = 16
NW = NC * NS
WIN = 128
E_PAD = 327680
EPT = E_PAD // NW
NWIN = EPT // WIN
N_PAD = 10112
RPT = N_PAD // NS
F2 = 16

_MESH = plsc.VectorSubcoreMesh(
    core_axis_name="c", subcore_axis_name="s", num_cores=NC, num_subcores=NS)


def _make_agg(F, name):
  NBUF = 4

  @functools.partial(
      pl.kernel,
      out_type=jax.ShapeDtypeStruct((NC, N_PAD, F), jnp.float32),
      mesh=_MESH,
      scratch_types=[
          pltpu.VMEM((NWIN, WIN), jnp.int32),
          pltpu.VMEM((NWIN, WIN), jnp.int32),
          [pltpu.VMEM((WIN, F), jnp.float32)] * NBUF,
          pltpu.VMEM_SHARED((N_PAD, F), jnp.float32),
          [pltpu.SemaphoreType.DMA] * NBUF,
          [pltpu.SemaphoreType.DMA] * NBUF,
      ],
      compiler_params=pltpu.CompilerParams(use_tc_tiling_on_sc=False),
      name=name,
  )
  def agg(y_hbm, srcw_hbm, dstw_hbm, z_hbm, out_hbm, src_i, dst_i, rows,
          acc_sh, gsem, ssem):
    c = lax.axis_index("c")
    s = lax.axis_index("s")
    wid = c * NS + s
    wbase = wid * NWIN
    pltpu.sync_copy(srcw_hbm.at[pl.ds(wbase, NWIN)], src_i)
    pltpu.sync_copy(dstw_hbm.at[pl.ds(wbase, NWIN)], dst_i)
    pltpu.sync_copy(z_hbm, acc_sh.at[pl.ds(s * RPT, RPT)])
    plsc.subcore_barrier()

    for b in range(NBUF):
      pltpu.async_copy(y_hbm.at[src_i.at[b]], rows[b], gsem[b])

    def grp(i, carry):
      w = NBUF * i
      for b in range(NBUF):
        pltpu.make_async_copy(y_hbm.at[src_i.at[w + b]], rows[b],
                              gsem[b]).wait()
        pltpu.async_copy(rows[b], acc_sh.at[dst_i.at[w + b]], ssem[b],
                         add=True)
      for b in range(NBUF):
        wn = w + NBUF + b

        @pl.when(wn < NWIN)
        def _():
          pltpu.make_async_copy(rows[b], acc_sh.at[dst_i.at[0]],
                                ssem[b]).wait()
          pltpu.async_copy(y_hbm.at[src_i.at[wn]], rows[b], gsem[b])

      return carry

    lax.fori_loop(0, NWIN // NBUF, grp, 0)
    for b in range(NBUF):
      pltpu.make_async_copy(rows[b], acc_sh.at[dst_i.at[0]], ssem[b]).wait()
    plsc.subcore_barrier()
    pltpu.sync_copy(acc_sh.at[pl.ds(s * RPT, RPT)],
                    out_hbm.at[c].at[pl.ds(s * RPT, RPT)])

  return agg


@functools.partial(
    pl.kernel,
    out_type=jax.ShapeDtypeStruct((NC, N_PAD, F2), jnp.float32),
    mesh=_MESH,
    scratch_types=[
        pltpu.VMEM((NWIN, WIN), jnp.int32),
        pltpu.VMEM((WIN, F2), jnp.float32),
        pltpu.VMEM_SHARED((N_PAD, F2), jnp.float32),
        pltpu.SemaphoreType.DMA,
    ],
    compiler_params=pltpu.CompilerParams(use_tc_tiling_on_sc=False),
    name="gcn_deg_sc",
)
def _deg_sc(dstw_hbm, ones_hbm, z_hbm, out_hbm, dst_i, ones_v, acc_sh, sem):
  c = lax.axis_index("c")
  s = lax.axis_index("s")
  wid = c * NS + s
  pltpu.sync_copy(dstw_hbm.at[pl.ds(wid * NWIN, NWIN)], dst_i)
  pltpu.sync_copy(ones_hbm, ones_v)
  pltpu.sync_copy(z_hbm, acc_sh.at[pl.ds(s * RPT, RPT)])
  plsc.subcore_barrier()

  def grp(g, carry):
    base = g * 8
    for j in range(8):
      pltpu.async_copy(ones_v, acc_sh.at[dst_i.at[base + j]], sem, add=True)
    for j in range(8):
      pltpu.make_async_copy(ones_v, acc_sh.at[dst_i.at[base + j]], sem).wait()
    return carry

  lax.fori_loop(0, NWIN // 8, grp, 0)
  plsc.subcore_barrier()
  pltpu.sync_copy(acc_sh.at[pl.ds(s * RPT, RPT)],
                  out_hbm.at[c].at[pl.ds(s * RPT, RPT)])


_agg64 = _make_agg(H, "gcn_agg64_sc")
_agg16 = _make_agg(F2, "gcn_agg16_sc")

_RB = 1000
_GRID = (N // _RB,)


def _tc1_body(x_ref, w1_ref, dp_ref, xw_ref, y1_ref, dinv_ref):
  deg = 1.0 + dp_ref[0] + dp_ref[1]
  dinv = lax.rsqrt(deg)
  xw = jnp.dot(x_ref[...], w1_ref[...], preferred_element_type=jnp.float32)
  d0 = dinv[:, 0:1]
  xw_ref[...] = xw
  y1_ref[...] = xw * d0
  dinv_ref[...] = dinv


_tc1 = pl.pallas_call(
    _tc1_body,
    grid=_GRID,
    in_specs=[
        pl.BlockSpec((_RB, D), lambda i: (i, 0)),
        pl.BlockSpec((D, H), lambda i: (0, 0)),
        pl.BlockSpec((NC, _RB, F2), lambda i: (0, i, 0)),
    ],
    out_specs=[
        pl.BlockSpec((_RB, H), lambda i: (i, 0)),
        pl.BlockSpec((_RB, H), lambda i: (i, 0)),
        pl.BlockSpec((_RB, F2), lambda i: (i, 0)),
    ],
    out_shape=[
        jax.ShapeDtypeStruct((N, H), jnp.float32),
        jax.ShapeDtypeStruct((N, H), jnp.float32),
        jax.ShapeDtypeStruct((N, F2), jnp.float32),
    ],
)


def _tc2_body(a1_ref, xw_ref, dinv_ref, b1_ref, w2_ref, hw2_ref, y2_ref):
  d0 = dinv_ref[:, 0:1]
  agg = a1_ref[0] + a1_ref[1]
  out1 = d0 * agg + (d0 * d0) * xw_ref[...] + b1_ref[...]
  h = jnp.maximum(out1, 0.0)
  hw2 = jnp.dot(h, w2_ref[...], preferred_element_type=jnp.float32)
  hw2_ref[...] = hw2
  y2_ref[...] = hw2 * d0


_tc2 = pl.pallas_call(
    _tc2_body,
    grid=_GRID,
    in_specs=[
        pl.BlockSpec((NC, _RB, H), lambda i: (0, i, 0)),
        pl.BlockSpec((_RB, H), lambda i: (i, 0)),
        pl.BlockSpec((_RB, F2), lambda i: (i, 0)),
        pl.BlockSpec((1, H), lambda i: (0, 0)),
        pl.BlockSpec((H, F2), lambda i: (0, 0)),
    ],
    out_specs=[
        pl.BlockSpec((_RB, F2), lambda i: (i, 0)),
        pl.BlockSpec((_RB, F2), lambda i: (i, 0)),
    ],
    out_shape=[
        jax.ShapeDtypeStruct((N, F2), jnp.float32),
        jax.ShapeDtypeStruct((N, F2), jnp.float32),
    ],
)


def _tc3_body(a2_ref, hw2_ref, dinv_ref, b2_ref, out_ref):
  d0 = dinv_ref[:, 0:1]
  agg = a2_ref[0] + a2_ref[1]
  out_ref[...] = d0 * agg + (d0 * d0) * hw2_ref[...] + b2_ref[...]


_tc3 = pl.pallas_call(
    _tc3_body,
    grid=_GRID,
    in_specs=[
        pl.BlockSpec((NC, _RB, F2), lambda i: (0, i, 0)),
        pl.BlockSpec((_RB, F2), lambda i: (i, 0)),
        pl.BlockSpec((_RB, F2), lambda i: (i, 0)),
        pl.BlockSpec((1, F2), lambda i: (0, 0)),
    ],
    out_specs=pl.BlockSpec((_RB, F2), lambda i: (i, 0)),
    out_shape=jax.ShapeDtypeStruct((N, F2), jnp.float32),
)


def kernel(x, edge_index, W1, b1, W2, b2):
  ei = edge_index.astype(jnp.int32)
  n_extra = E_PAD - E
  pad_src = (jnp.arange(n_extra, dtype=jnp.int32) % 9973)
  pad_dst = N + (jnp.arange(n_extra, dtype=jnp.int32) % (N_PAD - N))
  src = jnp.concatenate([ei[0], pad_src]).reshape(E_PAD // WIN, WIN)
  dst = jnp.concatenate([ei[1], pad_dst]).reshape(E_PAD // WIN, WIN)

  ones_w = jnp.ones((WIN, F2), jnp.float32)
  z16 = jnp.zeros((RPT, F2), jnp.float32)
  z64 = jnp.zeros((RPT, H), jnp.float32)

  dp = _deg_sc(dst, ones_w, z16)
  xw, y1, dinv = _tc1(x, W1, dp[:, :N])
  a1 = _agg64(y1, src, dst, z64)
  b1r = b1.reshape(1, H)
  w2p = jnp.zeros((H, F2), jnp.float32).at[:, :2].set(W2)
  hw2, y2 = _tc2(a1[:, :N], xw, dinv, b1r, w2p)
  a2 = _agg16(y2, src, dst, z16)
  b2p = jnp.zeros((1, F2), jnp.float32).at[0, :2].set(b2)
  out16 = _tc3(a2[:, :N], hw2, dinv, b2p)
  return out16[:, :2]


# speedup vs baseline: 49.3871x; 1.1847x over previous
import functools

import jax
import jax.numpy as jnp
from jax import lax
from jax.experimental import pallas as pl
from jax.experimental.pallas import tpu as pltpu
from jax.experimental.pallas import tpu_sc as plsc

N = 10000
E = 320000
D = 128
H = 64

NC = 2
NS = 16
NW = NC * NS
WIN = 128
E_PAD = 327680
EPT = E_PAD // NW
NWIN = EPT // WIN
N_PAD = 10112
RPT = N_PAD // NS
F2 = 16

_MESH = plsc.VectorSubcoreMesh(
    core_axis_name="c", subcore_axis_name="s", num_cores=NC, num_subcores=NS)


def _make_agg(F, name):
  NBUF = 8

  @functools.partial(
      pl.kernel,
      out_type=jax.ShapeDtypeStruct((NC, N_PAD, F), jnp.float32),
      mesh=_MESH,
      scratch_types=[
          pltpu.VMEM((NWIN, WIN), jnp.int32),
          pltpu.VMEM((NWIN, WIN), jnp.int32),
          [pltpu.VMEM((WIN, F), jnp.float32)] * NBUF,
          pltpu.VMEM_SHARED((N_PAD, F), jnp.float32),
          [pltpu.SemaphoreType.DMA] * NBUF,
          [pltpu.SemaphoreType.DMA] * NBUF,
      ],
      compiler_params=pltpu.CompilerParams(use_tc_tiling_on_sc=False),
      name=name,
  )
  def agg(y_hbm, srcw_hbm, dstw_hbm, z_hbm, out_hbm, src_i, dst_i, rows,
          acc_sh, gsem, ssem):
    c = lax.axis_index("c")
    s = lax.axis_index("s")
    wid = c * NS + s
    wbase = wid * NWIN
    pltpu.sync_copy(srcw_hbm.at[pl.ds(wbase, NWIN)], src_i)
    pltpu.sync_copy(dstw_hbm.at[pl.ds(wbase, NWIN)], dst_i)
    pltpu.sync_copy(z_hbm, acc_sh.at[pl.ds(s * RPT, RPT)])
    plsc.subcore_barrier()

    for b in range(NBUF):
      pltpu.async_copy(y_hbm.at[src_i.at[b]], rows[b], gsem[b])

    def grp(i, carry):
      w = NBUF * i
      for b in range(NBUF):
        pltpu.make_async_copy(y_hbm.at[src_i.at[w + b]], rows[b],
                              gsem[b]).wait()
        pltpu.async_copy(rows[b], acc_sh.at[dst_i.at[w + b]], ssem[b],
                         add=True)
      for b in range(NBUF):
        wn = w + NBUF + b

        @pl.when(wn < NWIN)
        def _():
          pltpu.make_async_copy(rows[b], acc_sh.at[dst_i.at[0]],
                                ssem[b]).wait()
          pltpu.async_copy(y_hbm.at[src_i.at[wn]], rows[b], gsem[b])

      return carry

    lax.fori_loop(0, NWIN // NBUF, grp, 0)
    for b in range(NBUF):
      pltpu.make_async_copy(rows[b], acc_sh.at[dst_i.at[0]], ssem[b]).wait()
    plsc.subcore_barrier()
    pltpu.sync_copy(acc_sh.at[pl.ds(s * RPT, RPT)],
                    out_hbm.at[c].at[pl.ds(s * RPT, RPT)])

  return agg


@functools.partial(
    pl.kernel,
    out_type=jax.ShapeDtypeStruct((NC, N_PAD, F2), jnp.float32),
    mesh=_MESH,
    scratch_types=[
        pltpu.VMEM((NWIN, WIN), jnp.int32),
        pltpu.VMEM((WIN, F2), jnp.float32),
        pltpu.VMEM_SHARED((N_PAD, F2), jnp.float32),
        pltpu.SemaphoreType.DMA,
    ],
    compiler_params=pltpu.CompilerParams(use_tc_tiling_on_sc=False),
    name="gcn_deg_sc",
)
def _deg_sc(dstw_hbm, ones_hbm, z_hbm, out_hbm, dst_i, ones_v, acc_sh, sem):
  c = lax.axis_index("c")
  s = lax.axis_index("s")
  wid = c * NS + s
  pltpu.sync_copy(dstw_hbm.at[pl.ds(wid * NWIN, NWIN)], dst_i)
  pltpu.sync_copy(ones_hbm, ones_v)
  pltpu.sync_copy(z_hbm, acc_sh.at[pl.ds(s * RPT, RPT)])
  plsc.subcore_barrier()

  def grp(g, carry):
    base = g * 8
    for j in range(8):
      pltpu.async_copy(ones_v, acc_sh.at[dst_i.at[base + j]], sem, add=True)
    for j in range(8):
      pltpu.make_async_copy(ones_v, acc_sh.at[dst_i.at[base + j]], sem).wait()
    return carry

  lax.fori_loop(0, NWIN // 8, grp, 0)
  plsc.subcore_barrier()
  pltpu.sync_copy(acc_sh.at[pl.ds(s * RPT, RPT)],
                  out_hbm.at[c].at[pl.ds(s * RPT, RPT)])


_agg64 = _make_agg(H, "gcn_agg64_sc")
_agg16 = _make_agg(F2, "gcn_agg16_sc")

_RB = 2000
_GRID = (N // _RB,)


def _tcmm1_body(x_ref, w1_ref, xw_ref):
  xw_ref[...] = jnp.dot(x_ref[...], w1_ref[...],
                        preferred_element_type=jnp.float32)


_tcmm1 = pl.pallas_call(
    _tcmm1_body,
    grid=_GRID,
    in_specs=[
        pl.BlockSpec((_RB, D), lambda i: (i, 0)),
        pl.BlockSpec((D, H), lambda i: (0, 0)),
    ],
    out_specs=pl.BlockSpec((_RB, H), lambda i: (i, 0)),
    out_shape=jax.ShapeDtypeStruct((N, H), jnp.float32),
)


def _tc1_body(xw_ref, dp_ref, y1_ref, dinv_ref):
  deg = 1.0 + dp_ref[0] + dp_ref[1]
  dinv = lax.rsqrt(deg)
  d0 = dinv[:, 0:1]
  y1_ref[...] = xw_ref[...] * d0
  dinv_ref[...] = dinv


_tc1 = pl.pallas_call(
    _tc1_body,
    grid=_GRID,
    in_specs=[
        pl.BlockSpec((_RB, H), lambda i: (i, 0)),
        pl.BlockSpec((NC, _RB, F2), lambda i: (0, i, 0)),
    ],
    out_specs=[
        pl.BlockSpec((_RB, H), lambda i: (i, 0)),
        pl.BlockSpec((_RB, F2), lambda i: (i, 0)),
    ],
    out_shape=[
        jax.ShapeDtypeStruct((N, H), jnp.float32),
        jax.ShapeDtypeStruct((N, F2), jnp.float32),
    ],
)


def _tc2_body(a1_ref, xw_ref, dinv_ref, b1_ref, w2_ref, hw2_ref, y2_ref):
  d0 = dinv_ref[:, 0:1]
  agg = a1_ref[0] + a1_ref[1]
  out1 = d0 * agg + (d0 * d0) * xw_ref[...] + b1_ref[...]
  h = jnp.maximum(out1, 0.0)
  hw2 = jnp.dot(h, w2_ref[...], preferred_element_type=jnp.float32)
  hw2_ref[...] = hw2
  y2_ref[...] = hw2 * d0


_tc2 = pl.pallas_call(
    _tc2_body,
    grid=_GRID,
    in_specs=[
        pl.BlockSpec((NC, _RB, H), lambda i: (0, i, 0)),
        pl.BlockSpec((_RB, H), lambda i: (i, 0)),
        pl.BlockSpec((_RB, F2), lambda i: (i, 0)),
        pl.BlockSpec((1, H), lambda i: (0, 0)),
        pl.BlockSpec((H, F2), lambda i: (0, 0)),
    ],
    out_specs=[
        pl.BlockSpec((_RB, F2), lambda i: (i, 0)),
        pl.BlockSpec((_RB, F2), lambda i: (i, 0)),
    ],
    out_shape=[
        jax.ShapeDtypeStruct((N, F2), jnp.float32),
        jax.ShapeDtypeStruct((N, F2), jnp.float32),
    ],
)


def _tc3_body(a2_ref, hw2_ref, dinv_ref, b2_ref, out_ref):
  d0 = dinv_ref[:, 0:1]
  agg = a2_ref[0] + a2_ref[1]
  out_ref[...] = d0 * agg + (d0 * d0) * hw2_ref[...] + b2_ref[...]


_tc3 = pl.pallas_call(
    _tc3_body,
    grid=_GRID,
    in_specs=[
        pl.BlockSpec((NC, _RB, F2), lambda i: (0, i, 0)),
        pl.BlockSpec((_RB, F2), lambda i: (i, 0)),
        pl.BlockSpec((_RB, F2), lambda i: (i, 0)),
        pl.BlockSpec((1, F2), lambda i: (0, 0)),
    ],
    out_specs=pl.BlockSpec((_RB, F2), lambda i: (i, 0)),
    out_shape=jax.ShapeDtypeStruct((N, F2), jnp.float32),
)


def kernel(x, edge_index, W1, b1, W2, b2):
  ei = edge_index.astype(jnp.int32)
  n_extra = E_PAD - E
  pad_src = (jnp.arange(n_extra, dtype=jnp.int32) % 9973)
  pad_dst = N + (jnp.arange(n_extra, dtype=jnp.int32) % (N_PAD - N))
  src = jnp.concatenate([ei[0], pad_src]).reshape(E_PAD // WIN, WIN)
  dst = jnp.concatenate([ei[1], pad_dst]).reshape(E_PAD // WIN, WIN)

  ones_w = jnp.ones((WIN, F2), jnp.float32)
  z16 = jnp.zeros((RPT, F2), jnp.float32)
  z64 = jnp.zeros((RPT, H), jnp.float32)

  dp = _deg_sc(dst, ones_w, z16)
  xw = _tcmm1(x, W1)
  y1, dinv = _tc1(xw, dp)
  a1 = _agg64(y1, src, dst, z64)
  b1r = b1.reshape(1, H)
  w2p = jnp.zeros((H, F2), jnp.float32).at[:, :2].set(W2)
  hw2, y2 = _tc2(a1, xw, dinv, b1r, w2p)
  a2 = _agg16(y2, src, dst, z16)
  b2p = jnp.zeros((1, F2), jnp.float32).at[0, :2].set(b2)
  out16 = _tc3(a2, hw2, dinv, b2p)
  return out16[:, :2]
